# Initial kernel scaffold; baseline (speedup 1.0000x reference)
import jax
import jax.numpy as jnp
from jax.experimental import pallas as pl


def kernel(x, edge_index, W1, b1, a1, W2, b2, a2, W3, b3, a3, Wl, bl):
    raise NotImplementedError("write your pallas kernel here")


# speedup vs baseline: 47.3552x; 47.3552x over previous
import functools

import jax
import jax.numpy as jnp
from jax import lax
from jax.experimental import pallas as pl
from jax.experimental.pallas import tpu as pltpu
from jax.experimental.pallas import tpu_sc as plsc

N = 10000
NP = 10240
NW = 32
SL = NP // NW
L = 16

_SC_PARAMS = pltpu.CompilerParams(needs_layout_passes=False,
                                  use_tc_tiling_on_sc=False)
_MESH = plsc.VectorSubcoreMesh(core_axis_name="c", subcore_axis_name="s",
                               num_cores=2, num_subcores=16)


def _wid():
    return lax.axis_index("s") * 2 + lax.axis_index("c")


def _bcast(v):
    return jnp.broadcast_to(v.reshape(-1)[:, None], (v.size, L)).astype(jnp.float32)


def _mm1(xp, w1p):
    def body(w_ref, x_ref, o_ref):
        o_ref[...] = lax.dot_general(
            w_ref[...], x_ref[...], (((1,), (1,)), ((), ())),
            preferred_element_type=jnp.float32)

    return pl.pallas_call(
        body,
        grid=(NP // 512,),
        in_specs=[pl.BlockSpec((8, 128), lambda i: (0, 0)),
                  pl.BlockSpec((512, 128), lambda i: (i, 0))],
        out_specs=pl.BlockSpec((8, 512), lambda i: (0, i)),
        out_shape=jax.ShapeDtypeStruct((8, NP), jnp.float32),
    )(w1p, xp)


def _deg(dst, ch):
    @functools.partial(
        pl.kernel,
        out_type=jax.ShapeDtypeStruct((NW, NP), jnp.float32),
        mesh=_MESH,
        compiler_params=_SC_PARAMS,
        scratch_types=[pltpu.VMEM((ch,), jnp.int32),
                       pltpu.VMEM((NP,), jnp.float32)])
    def k(dst_hbm, degp_hbm, dst_v, acc_v):
        w = _wid()
        pltpu.sync_copy(dst_hbm.at[pl.ds(w * ch, ch)], dst_v)

        @pl.loop(0, NP // L)
        def _z(i):
            acc_v[pl.ds(i * L, L)] = jnp.zeros((L,), jnp.float32)

        ones = jnp.ones((L,), jnp.float32)

        @pl.loop(0, ch // L)
        def _e(e):
            idx = dst_v[pl.ds(e * L, L)]
            plsc.addupdate_scatter(acc_v, [idx], ones)

        pltpu.sync_copy(acc_v, degp_hbm.at[w])

    return k(dst)


def _rsqrt16(s):
    xhalf = 0.5 * s
    i = plsc.bitcast(s, jnp.int32)
    i = jnp.int32(0x5F3759DF) - lax.shift_right_logical(i, 1)
    y = plsc.bitcast(i, jnp.float32)
    y = y * (1.5 - xhalf * y * y)
    y = y * (1.5 - xhalf * y * y)
    y = y * (1.5 - xhalf * y * y)
    return y


def _dis(degp):
    @functools.partial(
        pl.kernel,
        out_type=jax.ShapeDtypeStruct((NP,), jnp.float32),
        mesh=_MESH,
        compiler_params=_SC_PARAMS,
        scratch_types=[pltpu.VMEM((NW, SL), jnp.float32),
                       pltpu.VMEM((SL,), jnp.float32)])
    def k(degp_hbm, dis_hbm, part_v, dis_v):
        w = _wid()
        pltpu.sync_copy(degp_hbm.at[:, pl.ds(w * SL, SL)], part_v)

        @pl.loop(0, SL // L)
        def _g(g):
            sl = pl.ds(g * L, L)
            s = part_v[0, sl]
            for p in range(1, NW):
                s = s + part_v[p, sl]
            dis_v[sl] = _rsqrt16(s)

        pltpu.sync_copy(dis_v, dis_hbm.at[pl.ds(w * SL, SL)])

    return k(degp)


def _norm(dis, src, dst, ch):
    etp = NW * ch

    @functools.partial(
        pl.kernel,
        out_type=jax.ShapeDtypeStruct((etp,), jnp.float32),
        mesh=_MESH,
        compiler_params=_SC_PARAMS,
        scratch_types=[pltpu.VMEM((NP,), jnp.float32),
                       pltpu.VMEM((ch,), jnp.int32),
                       pltpu.VMEM((ch,), jnp.int32),
                       pltpu.VMEM((ch,), jnp.float32)])
    def k(dis_hbm, src_hbm, dst_hbm, norm_hbm, dis_v, src_v, dst_v, norm_v):
        w = _wid()
        pltpu.sync_copy(dis_hbm, dis_v)
        pltpu.sync_copy(src_hbm.at[pl.ds(w * ch, ch)], src_v)
        pltpu.sync_copy(dst_hbm.at[pl.ds(w * ch, ch)], dst_v)

        @pl.loop(0, ch // L)
        def _e(e):
            sl = pl.ds(e * L, L)
            norm_v[sl] = (plsc.load_gather(dis_v, [src_v[sl]]) *
                          plsc.load_gather(dis_v, [dst_v[sl]]))

        pltpu.sync_copy(norm_v, norm_hbm.at[pl.ds(w * ch, ch)])

    return k(dis, src, dst)


def _conv(xw, src, dst, norm, d, ch):
    @functools.partial(
        pl.kernel,
        out_type=jax.ShapeDtypeStruct((NW, d, NP), jnp.float32),
        mesh=_MESH,
        compiler_params=_SC_PARAMS,
        scratch_types=[pltpu.VMEM((d, NP), jnp.float32),
                       pltpu.VMEM((d, NP), jnp.float32),
                       pltpu.VMEM((ch,), jnp.int32),
                       pltpu.VMEM((ch,), jnp.int32),
                       pltpu.VMEM((ch,), jnp.float32)])
    def k(xw_hbm, src_hbm, dst_hbm, norm_hbm, part_hbm,
          xw_v, acc_v, src_v, dst_v, norm_v):
        w = _wid()
        pltpu.sync_copy(xw_hbm.at[pl.ds(0, d)], xw_v)
        pltpu.sync_copy(src_hbm.at[pl.ds(w * ch, ch)], src_v)
        pltpu.sync_copy(dst_hbm.at[pl.ds(w * ch, ch)], dst_v)
        pltpu.sync_copy(norm_hbm.at[pl.ds(w * ch, ch)], norm_v)

        def zero_row(j):
            @pl.loop(0, NP // L)
            def _z(i):
                acc_v[j, pl.ds(i * L, L)] = jnp.zeros((L,), jnp.float32)

        for j in range(d):
            zero_row(j)

        jvec = [jnp.full((L,), j, jnp.int32) for j in range(d)]

        @pl.loop(0, ch // L)
        def _e(e):
            sl = pl.ds(e * L, L)
            s_idx = src_v[sl]
            d_idx = dst_v[sl]
            nrm = norm_v[sl]
            for j in range(d):
                v = plsc.load_gather(xw_v, [jvec[j], s_idx]) * nrm
                plsc.addupdate_scatter(acc_v, [jvec[j], d_idx], v)

        pltpu.sync_copy(acc_v, part_hbm.at[w])

    return k(xw, src, dst, norm)


def _transform(part, bb, ab, wb, din, dout):
    @functools.partial(
        pl.kernel,
        out_type=jax.ShapeDtypeStruct((dout, NP), jnp.float32),
        mesh=_MESH,
        compiler_params=_SC_PARAMS,
        scratch_types=[pltpu.VMEM((NW, din, SL), jnp.float32),
                       pltpu.VMEM((dout, SL), jnp.float32),
                       pltpu.VMEM((din, L), jnp.float32),
                       pltpu.VMEM((din, L), jnp.float32),
                       pltpu.VMEM((din * dout, L), jnp.float32)])
    def k(part_hbm, bb_hbm, ab_hbm, wb_hbm, xw_hbm,
          part_v, xw_v, bb_v, ab_v, wb_v):
        w = _wid()
        pltpu.sync_copy(part_hbm.at[:, :, pl.ds(w * SL, SL)], part_v)
        pltpu.sync_copy(bb_hbm, bb_v)
        pltpu.sync_copy(ab_hbm, ab_v)
        pltpu.sync_copy(wb_hbm, wb_v)

        @pl.loop(0, SL // L)
        def _g(g):
            sl = pl.ds(g * L, L)
            hs = []
            for kf in range(din):
                acc = part_v[0, kf, sl]
                for p in range(1, NW):
                    acc = acc + part_v[p, kf, sl]
                acc = acc + bb_v[kf]
                acc = jnp.where(acc >= 0.0, acc, ab_v[kf] * acc)
                hs.append(acc)
            for j in range(dout):
                o = hs[0] * wb_v[j]
                for kf in range(1, din):
                    o = o + hs[kf] * wb_v[kf * dout + j]
                xw_v[j, sl] = o

        pltpu.sync_copy(xw_v, xw_hbm.at[:, pl.ds(w * SL, SL)])

    return k(part, bb, ab, wb)


def _final(part, bb, ab, wlb, blb):
    @functools.partial(
        pl.kernel,
        out_type=(jax.ShapeDtypeStruct((NP, 16), jnp.float32),
                  jax.ShapeDtypeStruct((NP, 2), jnp.float32)),
        mesh=_MESH,
        compiler_params=_SC_PARAMS,
        scratch_types=[pltpu.VMEM((NW, 2, SL), jnp.float32),
                       pltpu.VMEM((SL, 16), jnp.float32),
                       pltpu.VMEM((SL, 2), jnp.float32),
                       pltpu.VMEM((2, L), jnp.float32),
                       pltpu.VMEM((2, L), jnp.float32),
                       pltpu.VMEM((32, L), jnp.float32),
                       pltpu.VMEM((16, L), jnp.float32)])
    def k(part_hbm, bb_hbm, ab_hbm, wlb_hbm, blb_hbm, out_hbm, h_hbm,
          part_v, out_v, h_v, bb_v, ab_v, wlb_v, blb_v):
        w = _wid()
        pltpu.sync_copy(part_hbm.at[:, :, pl.ds(w * SL, SL)], part_v)
        pltpu.sync_copy(bb_hbm, bb_v)
        pltpu.sync_copy(ab_hbm, ab_v)
        pltpu.sync_copy(wlb_hbm, wlb_v)
        pltpu.sync_copy(blb_hbm, blb_v)

        lane = lax.iota(jnp.int32, L)
        jvec2 = [jnp.full((L,), j, jnp.int32) for j in range(2)]
        jvec16 = [jnp.full((L,), j, jnp.int32) for j in range(16)]

        @pl.loop(0, SL // L)
        def _g(g):
            sl = pl.ds(g * L, L)
            rows = lane + g * L
            hs = []
            for kf in range(2):
                acc = part_v[0, kf, sl]
                for p in range(1, NW):
                    acc = acc + part_v[p, kf, sl]
                acc = acc + bb_v[kf]
                acc = jnp.where(acc >= 0.0, acc, ab_v[kf] * acc)
                hs.append(acc)
                plsc.store_scatter(h_v, [rows, jvec2[kf]], acc)
            for j in range(16):
                o = hs[0] * wlb_v[j] + hs[1] * wlb_v[16 + j] + blb_v[j]
                plsc.store_scatter(out_v, [rows, jvec16[j]], o)

        pltpu.sync_copy(out_v, out_hbm.at[pl.ds(w * SL, SL)])
        pltpu.sync_copy(h_v, h_hbm.at[pl.ds(w * SL, SL)])

    return k(part, bb, ab, wlb, blb)


def kernel(x, edge_index, W1, b1, a1, W2, b2, a2, W3, b3, a3, Wl, bl):
    e = edge_index.shape[1]
    et = e + N
    ch = -(-et // (NW * L)) * L
    etp = NW * ch
    pad = etp - et

    loop = jnp.arange(N, dtype=jnp.int32)
    padv = jnp.full((pad,), N, jnp.int32)
    src = jnp.concatenate([edge_index[0].astype(jnp.int32), loop, padv])
    dst = jnp.concatenate([edge_index[1].astype(jnp.int32), loop, padv])

    xp = jnp.pad(x, ((0, NP - N), (0, 0)))
    w1p = jnp.zeros((8, x.shape[1]), jnp.float32).at[:4].set(W1.T)

    xw1 = _mm1(xp, w1p)
    degp = _deg(dst, ch)
    dis = _dis(degp)
    nrm = _norm(dis, src, dst, ch)

    p1 = _conv(xw1, src, dst, nrm, 4, ch)
    xw2 = _transform(p1, _bcast(b1), _bcast(a1), _bcast(W2), 4, 4)
    p2 = _conv(xw2, src, dst, nrm, 4, ch)
    xw3 = _transform(p2, _bcast(b2), _bcast(a2), _bcast(W3), 4, 2)
    p3 = _conv(xw3, src, dst, nrm, 2, ch)
    out_p, h_p = _final(p3, _bcast(b3), _bcast(a3), _bcast(Wl), _bcast(bl))

    return out_p[:N], h_p[:N]


# speedup vs baseline: 81.5031x; 1.7211x over previous
import functools

import jax
import jax.numpy as jnp
from jax import lax
from jax.experimental import pallas as pl
from jax.experimental.pallas import tpu as pltpu
from jax.experimental.pallas import tpu_sc as plsc

N = 10000
NP = 10240
NW = 32
SL = NP // NW
L = 16

_SC_PARAMS = pltpu.CompilerParams(needs_layout_passes=False,
                                  use_tc_tiling_on_sc=False)
_MESH = plsc.VectorSubcoreMesh(core_axis_name="c", subcore_axis_name="s",
                               num_cores=2, num_subcores=16)


def _wid():
    return lax.axis_index("s") * 2 + lax.axis_index("c")


def _bcast(v):
    return jnp.broadcast_to(v.reshape(-1)[:, None], (v.size, L)).astype(jnp.float32)


def _mm1(xp, w1p):
    def body(w_ref, x_ref, o_ref):
        o_ref[...] = lax.dot_general(
            w_ref[...], x_ref[...], (((1,), (1,)), ((), ())),
            preferred_element_type=jnp.float32)

    return pl.pallas_call(
        body,
        grid=(NP // 512,),
        in_specs=[pl.BlockSpec((8, 128), lambda i: (0, 0)),
                  pl.BlockSpec((512, 128), lambda i: (i, 0))],
        out_specs=pl.BlockSpec((8, 512), lambda i: (0, i)),
        out_shape=jax.ShapeDtypeStruct((8, NP), jnp.float32),
    )(w1p, xp)


def _deg(dst, ch):
    @functools.partial(
        pl.kernel,
        out_type=jax.ShapeDtypeStruct((NW, NP), jnp.float32),
        mesh=_MESH,
        compiler_params=_SC_PARAMS,
        scratch_types=[pltpu.VMEM((ch,), jnp.int32),
                       pltpu.VMEM((NP,), jnp.float32),
                       pltpu.SemaphoreType.DMA])
    def k(dst_hbm, degp_hbm, dst_v, acc_v, sem):
        w = _wid()
        cp = pltpu.async_copy(dst_hbm.at[pl.ds(w * ch, ch)], dst_v, sem)

        zeros = jnp.zeros((L,), jnp.float32)

        @functools.partial(plsc.parallel_loop, 0, NP // L, unroll=8)
        def _z(i):
            acc_v[pl.ds(i * L, L)] = zeros

        cp.wait()
        ones = jnp.ones((L,), jnp.float32)

        @functools.partial(plsc.parallel_loop, 0, ch // L, unroll=4)
        def _e(e):
            idx = dst_v[pl.ds(e * L, L)]
            plsc.addupdate_scatter(acc_v, [idx], ones)

        pltpu.sync_copy(acc_v, degp_hbm.at[w])

    return k(dst)


def _rsqrt16(s):
    xhalf = 0.5 * s
    i = plsc.bitcast(s, jnp.int32)
    i = jnp.int32(0x5F3759DF) - lax.shift_right_logical(i, 1)
    y = plsc.bitcast(i, jnp.float32)
    y = y * (1.5 - xhalf * y * y)
    y = y * (1.5 - xhalf * y * y)
    y = y * (1.5 - xhalf * y * y)
    return y


def _dis(degp):
    @functools.partial(
        pl.kernel,
        out_type=jax.ShapeDtypeStruct((NP,), jnp.float32),
        mesh=_MESH,
        compiler_params=_SC_PARAMS,
        scratch_types=[pltpu.VMEM((NW, SL), jnp.float32),
                       pltpu.VMEM((SL,), jnp.float32)])
    def k(degp_hbm, dis_hbm, part_v, dis_v):
        w = _wid()
        pltpu.sync_copy(degp_hbm.at[:, pl.ds(w * SL, SL)], part_v)

        @functools.partial(plsc.parallel_loop, 0, SL // L, unroll=2)
        def _g(g):
            sl = pl.ds(g * L, L)
            s = part_v[0, sl]
            for p in range(1, NW):
                s = s + part_v[p, sl]
            dis_v[sl] = _rsqrt16(s)

        pltpu.sync_copy(dis_v, dis_hbm.at[pl.ds(w * SL, SL)])

    return k(degp)


def _norm(dis, src, dst, ch):
    etp = NW * ch

    @functools.partial(
        pl.kernel,
        out_type=jax.ShapeDtypeStruct((etp,), jnp.float32),
        mesh=_MESH,
        compiler_params=_SC_PARAMS,
        scratch_types=[pltpu.VMEM((NP,), jnp.float32),
                       pltpu.VMEM((ch,), jnp.int32),
                       pltpu.VMEM((ch,), jnp.int32),
                       pltpu.VMEM((ch,), jnp.float32),
                       pltpu.SemaphoreType.DMA])
    def k(dis_hbm, src_hbm, dst_hbm, norm_hbm, dis_v, src_v, dst_v, norm_v,
          sem):
        w = _wid()
        c1 = pltpu.async_copy(dis_hbm, dis_v, sem)
        c2 = pltpu.async_copy(src_hbm.at[pl.ds(w * ch, ch)], src_v, sem)
        c3 = pltpu.async_copy(dst_hbm.at[pl.ds(w * ch, ch)], dst_v, sem)
        c1.wait(); c2.wait(); c3.wait()

        @functools.partial(plsc.parallel_loop, 0, ch // L, unroll=4)
        def _e(e):
            sl = pl.ds(e * L, L)
            norm_v[sl] = (plsc.load_gather(dis_v, [src_v[sl]]) *
                          plsc.load_gather(dis_v, [dst_v[sl]]))

        pltpu.sync_copy(norm_v, norm_hbm.at[pl.ds(w * ch, ch)])

    return k(dis, src, dst)


def _conv(xw, src, dst, norm, d, ch):
    @functools.partial(
        pl.kernel,
        out_type=jax.ShapeDtypeStruct((NW, d, NP), jnp.float32),
        mesh=_MESH,
        compiler_params=_SC_PARAMS,
        scratch_types=[pltpu.VMEM((d, NP), jnp.float32),
                       pltpu.VMEM((d, NP), jnp.float32),
                       pltpu.VMEM((ch,), jnp.int32),
                       pltpu.VMEM((ch,), jnp.int32),
                       pltpu.VMEM((ch,), jnp.float32),
                       pltpu.SemaphoreType.DMA])
    def k(xw_hbm, src_hbm, dst_hbm, norm_hbm, part_hbm,
          xw_v, acc_v, src_v, dst_v, norm_v, sem):
        w = _wid()
        c1 = pltpu.async_copy(xw_hbm.at[pl.ds(0, d)], xw_v, sem)
        c2 = pltpu.async_copy(src_hbm.at[pl.ds(w * ch, ch)], src_v, sem)
        c3 = pltpu.async_copy(dst_hbm.at[pl.ds(w * ch, ch)], dst_v, sem)
        c4 = pltpu.async_copy(norm_hbm.at[pl.ds(w * ch, ch)], norm_v, sem)

        zeros = jnp.zeros((L,), jnp.float32)

        @functools.partial(plsc.parallel_loop, 0, NP // L, unroll=8)
        def _z(i):
            sl = pl.ds(i * L, L)
            for j in range(d):
                acc_v[j, sl] = zeros

        c1.wait(); c2.wait(); c3.wait(); c4.wait()

        jvec = [jnp.full((L,), j, jnp.int32) for j in range(d)]

        @functools.partial(plsc.parallel_loop, 0, ch // L, unroll=4)
        def _e(e):
            sl = pl.ds(e * L, L)
            s_idx = src_v[sl]
            d_idx = dst_v[sl]
            nrm = norm_v[sl]
            for j in range(d):
                v = plsc.load_gather(xw_v, [jvec[j], s_idx]) * nrm
                plsc.addupdate_scatter(acc_v, [jvec[j], d_idx], v)

        pltpu.sync_copy(acc_v, part_hbm.at[w])

    return k(xw, src, dst, norm)


def _transform(part, bb, ab, wb, din, dout):
    @functools.partial(
        pl.kernel,
        out_type=jax.ShapeDtypeStruct((dout, NP), jnp.float32),
        mesh=_MESH,
        compiler_params=_SC_PARAMS,
        scratch_types=[pltpu.VMEM((NW, din, SL), jnp.float32),
                       pltpu.VMEM((dout, SL), jnp.float32),
                       pltpu.VMEM((din, L), jnp.float32),
                       pltpu.VMEM((din, L), jnp.float32),
                       pltpu.VMEM((din * dout, L), jnp.float32),
                       pltpu.SemaphoreType.DMA])
    def k(part_hbm, bb_hbm, ab_hbm, wb_hbm, xw_hbm,
          part_v, xw_v, bb_v, ab_v, wb_v, sem):
        w = _wid()
        c1 = pltpu.async_copy(part_hbm.at[:, :, pl.ds(w * SL, SL)], part_v, sem)
        c2 = pltpu.async_copy(bb_hbm, bb_v, sem)
        c3 = pltpu.async_copy(ab_hbm, ab_v, sem)
        c4 = pltpu.async_copy(wb_hbm, wb_v, sem)
        c1.wait(); c2.wait(); c3.wait(); c4.wait()

        @functools.partial(plsc.parallel_loop, 0, SL // L, unroll=2)
        def _g(g):
            sl = pl.ds(g * L, L)
            hs = []
            for kf in range(din):
                acc = part_v[0, kf, sl]
                for p in range(1, NW):
                    acc = acc + part_v[p, kf, sl]
                acc = acc + bb_v[kf]
                acc = jnp.where(acc >= 0.0, acc, ab_v[kf] * acc)
                hs.append(acc)
            for j in range(dout):
                o = hs[0] * wb_v[j]
                for kf in range(1, din):
                    o = o + hs[kf] * wb_v[kf * dout + j]
                xw_v[j, sl] = o

        pltpu.sync_copy(xw_v, xw_hbm.at[:, pl.ds(w * SL, SL)])

    return k(part, bb, ab, wb)


def _final(part, bb, ab, wlb, blb):
    @functools.partial(
        pl.kernel,
        out_type=(jax.ShapeDtypeStruct((NP, 16), jnp.float32),
                  jax.ShapeDtypeStruct((NP, 2), jnp.float32)),
        mesh=_MESH,
        compiler_params=_SC_PARAMS,
        scratch_types=[pltpu.VMEM((NW, 2, SL), jnp.float32),
                       pltpu.VMEM((SL, 16), jnp.float32),
                       pltpu.VMEM((SL, 2), jnp.float32),
                       pltpu.VMEM((2, L), jnp.float32),
                       pltpu.VMEM((2, L), jnp.float32),
                       pltpu.VMEM((32, L), jnp.float32),
                       pltpu.VMEM((16, L), jnp.float32),
                       pltpu.SemaphoreType.DMA])
    def k(part_hbm, bb_hbm, ab_hbm, wlb_hbm, blb_hbm, out_hbm, h_hbm,
          part_v, out_v, h_v, bb_v, ab_v, wlb_v, blb_v, sem):
        w = _wid()
        c1 = pltpu.async_copy(part_hbm.at[:, :, pl.ds(w * SL, SL)], part_v, sem)
        c2 = pltpu.async_copy(bb_hbm, bb_v, sem)
        c3 = pltpu.async_copy(ab_hbm, ab_v, sem)
        c4 = pltpu.async_copy(wlb_hbm, wlb_v, sem)
        c5 = pltpu.async_copy(blb_hbm, blb_v, sem)
        c1.wait(); c2.wait(); c3.wait(); c4.wait(); c5.wait()

        lane = lax.iota(jnp.int32, L)
        jvec2 = [jnp.full((L,), j, jnp.int32) for j in range(2)]
        jvec16 = [jnp.full((L,), j, jnp.int32) for j in range(16)]

        @functools.partial(plsc.parallel_loop, 0, SL // L, unroll=2)
        def _g(g):
            sl = pl.ds(g * L, L)
            rows = lane + g * L
            hs = []
            for kf in range(2):
                acc = part_v[0, kf, sl]
                for p in range(1, NW):
                    acc = acc + part_v[p, kf, sl]
                acc = acc + bb_v[kf]
                acc = jnp.where(acc >= 0.0, acc, ab_v[kf] * acc)
                hs.append(acc)
                plsc.store_scatter(h_v, [rows, jvec2[kf]], acc)
            for j in range(16):
                o = hs[0] * wlb_v[j] + hs[1] * wlb_v[16 + j] + blb_v[j]
                plsc.store_scatter(out_v, [rows, jvec16[j]], o)

        pltpu.sync_copy(out_v, out_hbm.at[pl.ds(w * SL, SL)])
        pltpu.sync_copy(h_v, h_hbm.at[pl.ds(w * SL, SL)])

    return k(part, bb, ab, wlb, blb)


def kernel(x, edge_index, W1, b1, a1, W2, b2, a2, W3, b3, a3, Wl, bl):
    e = edge_index.shape[1]
    et = e + N
    ch = -(-et // (NW * L * 8)) * (L * 8)
    etp = NW * ch
    pad = etp - et

    loop = jnp.arange(N, dtype=jnp.int32)
    padv = jnp.full((pad,), N, jnp.int32)
    src = jnp.concatenate([edge_index[0].astype(jnp.int32), loop, padv])
    dst = jnp.concatenate([edge_index[1].astype(jnp.int32), loop, padv])

    xp = jnp.pad(x, ((0, NP - N), (0, 0)))
    w1p = jnp.zeros((8, x.shape[1]), jnp.float32).at[:4].set(W1.T)

    xw1 = _mm1(xp, w1p)
    degp = _deg(dst, ch)
    dis = _dis(degp)
    nrm = _norm(dis, src, dst, ch)

    p1 = _conv(xw1, src, dst, nrm, 4, ch)
    xw2 = _transform(p1, _bcast(b1), _bcast(a1), _bcast(W2), 4, 4)
    p2 = _conv(xw2, src, dst, nrm, 4, ch)
    xw3 = _transform(p2, _bcast(b2), _bcast(a2), _bcast(W3), 4, 2)
    p3 = _conv(xw3, src, dst, nrm, 2, ch)
    out_p, h_p = _final(p3, _bcast(b3), _bcast(a3), _bcast(Wl), _bcast(bl))

    return out_p[:N], h_p[:N]
